# baseline (speedup 1.0000x reference)
import functools

import jax
import jax.numpy as jnp
import numpy as np
from jax import lax
from jax.experimental import pallas as pl
from jax.experimental.pallas import tpu as pltpu
from jax.experimental.pallas import tpu_sc as plsc

N_ATOM = 256
HALF = 128
N_RBF = 16
CUTOFF = 10.0
N_NODES = 10000
N_EDGES = 160000
NUM_CONV = 3

NC = 2
NS = 16
K = 80
EPT = N_EDGES // NS
NCHUNK = EPT // K
RPT = 624
REM_R = N_NODES - NS * RPT

_MU = np.linspace(np.exp(-CUTOFF), 1.0, N_RBF).astype(np.float32)
_BETA = np.float32((2.0 / N_RBF * (1.0 - np.exp(-CUTOFF))) ** (-2))

_sc_mesh = plsc.VectorSubcoreMesh(core_axis_name="c", subcore_axis_name="s")


@functools.partial(
    pl.kernel,
    out_type=jax.ShapeDtypeStruct((2, N_EDGES, 16), jnp.float32),
    mesh=_sc_mesh,
    compiler_params=pltpu.CompilerParams(use_tc_tiling_on_sc=False),
    scratch_types=[
        pltpu.VMEM((NCHUNK, K), jnp.int32),
        pltpu.VMEM((K, 16), jnp.float32),
        pltpu.SemaphoreType.DMA,
    ],
)
def _geom_gather_kernel(xyz_hbm, idx_hbm, out_hbm, idx_v, rows_v, sem):
    c = lax.axis_index("c")
    s = lax.axis_index("s")
    pltpu.sync_copy(idx_hbm.at[c, s], idx_v)

    def chunk(j, carry):
        pltpu.async_copy(xyz_hbm.at[idx_v.at[j]], rows_v, sem).wait()
        base = s * EPT + j * K
        pltpu.sync_copy(rows_v, out_hbm.at[c, pl.ds(base, K)])
        return carry

    lax.fori_loop(0, NCHUNK, chunk, 0)


def _make_edge_conv(ci):
  @functools.partial(
      pl.kernel,
      out_type=jax.ShapeDtypeStruct((2, N_NODES, HALF), jnp.float32),
      mesh=_sc_mesh,
      scratch_types=[
          pltpu.VMEM_SHARED((N_NODES, HALF), jnp.float32),
          [pltpu.VMEM((K,), jnp.int32)] * 2,
          [pltpu.VMEM((K,), jnp.int32)] * 2,
          [pltpu.VMEM((K, HALF), jnp.float32)] * 2,
          [pltpu.VMEM((K, HALF), jnp.float32)] * 2,
          [pltpu.SemaphoreType.DMA] * 2,
          [pltpu.SemaphoreType.DMA] * 2,
          [pltpu.SemaphoreType.DMA] * 2,
          [pltpu.SemaphoreType.DMA] * 2,
          [pltpu.SemaphoreType.DMA] * 2,
      ],
  )
  def _edge_conv_kernel(phi_hbm, w3_hbm, src_hbm, dst_hbm, out_hbm,
                        acc, sidx, didx, rows, wrow,
                        gsem, wsem, scsem, sisem, disem):
    w_hbm = w3_hbm.at[ci]
    c = lax.axis_index("c")
    s = lax.axis_index("s")
    base0 = s * EPT

    zero16 = jnp.zeros((16,), jnp.float32)

    def zrow(i, carry):
        for jj in range(HALF // 16):
            rows[0][i, pl.ds(jj * 16, 16)] = zero16
        return carry

    lax.fori_loop(0, K, zrow, 0)
    base_r = s * RPT
    nfull = RPT // K
    rem = RPT - nfull * K
    for t in range(nfull):
        pltpu.sync_copy(rows[0], acc.at[pl.ds(base_r + t * K, K)])
    if rem:
        pltpu.sync_copy(rows[0].at[pl.ds(0, rem)],
                        acc.at[pl.ds(base_r + nfull * K, rem)])

    @pl.when(s == NS - 1)
    def _zero_tail():
        pltpu.sync_copy(rows[0].at[pl.ds(0, REM_R)],
                        acc.at[pl.ds(NS * RPT, REM_R)])

    plsc.subcore_barrier()

    def _issue_sidx(j, p):
        jc = jnp.minimum(j, NCHUNK - 1)
        pltpu.async_copy(src_hbm.at[pl.ds(base0 + jc * K, K)],
                         sidx[p], sisem[p])

    def _issue_didx(j, p):
        jc = jnp.minimum(j, NCHUNK - 1)
        pltpu.async_copy(dst_hbm.at[pl.ds(base0 + jc * K, K)],
                         didx[p], disem[p])

    def _issue_gather(p):
        pltpu.async_copy(phi_hbm.at[c].at[sidx[p]], rows[p], gsem[p])

    def _issue_w(j, p):
        pltpu.async_copy(w_hbm.at[c, pl.ds(base0 + j * K, K)],
                         wrow[p], wsem[p])

    def _wait(src, dst, sem):
        pltpu.make_async_copy(src, dst, sem).wait()

    def _multiply(p):
        @plsc.parallel_loop(0, K, step=1, unroll=4)
        def mul(i):
            for jj in range(HALF // 16):
                sl = pl.ds(jj * 16, 16)
                rows[p][i, sl] = rows[p][i, sl] * wrow[p][i, sl]

    def _full_iter(j, p):
        q = 1 - p
        _wait(phi_hbm.at[c].at[sidx[p]], rows[p], gsem[p])
        _wait(w_hbm.at[c, pl.ds(0, K)], wrow[p], wsem[p])

        @pl.when(j >= 1)
        def _wait_prev_scatter():
            _wait(rows[q], acc.at[didx[q]], scsem[q])

        _issue_didx(j + 1, q)
        _wait(src_hbm.at[pl.ds(0, K)], sidx[q], sisem[q])
        _issue_gather(q)
        _issue_w(j + 1, q)
        _issue_sidx(j + 2, p)
        _multiply(p)
        _wait(dst_hbm.at[pl.ds(0, K)], didx[p], disem[p])
        pltpu.async_copy(rows[p], acc.at[didx[p]], scsem[p], add=True)

    _issue_sidx(0, 0)
    _issue_didx(0, 0)
    _issue_sidx(1, 1)
    _wait(src_hbm.at[pl.ds(0, K)], sidx[0], sisem[0])
    _issue_gather(0)
    _issue_w(0, 0)

    def pair(j2, carry):
        j = j2 * 2
        _full_iter(j, 0)
        _full_iter(j + 1, 1)
        return carry

    lax.fori_loop(0, (NCHUNK - 1) // 2, pair, 0)

    _wait(phi_hbm.at[c].at[sidx[0]], rows[0], gsem[0])
    _wait(w_hbm.at[c, pl.ds(0, K)], wrow[0], wsem[0])
    _multiply(0)
    _wait(rows[1], acc.at[didx[1]], scsem[1])
    _wait(dst_hbm.at[pl.ds(0, K)], didx[0], disem[0])
    pltpu.async_copy(rows[0], acc.at[didx[0]], scsem[0], add=True)
    _wait(rows[0], acc.at[didx[0]], scsem[0])
    _wait(src_hbm.at[pl.ds(0, K)], sidx[1], sisem[1])
    plsc.subcore_barrier()
    pltpu.sync_copy(acc.at[pl.ds(base_r, RPT)],
                    out_hbm.at[c, pl.ds(base_r, RPT)])

    @pl.when(s == NS - 1)
    def _copy_tail():
        pltpu.sync_copy(acc.at[pl.ds(NS * RPT, REM_R)],
                        out_hbm.at[c, pl.ds(NS * RPT, REM_R)])

  return _edge_conv_kernel


_EDGE_CONVS = [_make_edge_conv(i) for i in range(NUM_CONV)]


_RB = 4000
_NB = 1000


def _silu(x):
    return x * jax.nn.sigmoid(x)


def _edge_basis_body(g_ref, out_ref):
    r = g_ref[0, :, :3] - g_ref[1, :, :3]
    d = jnp.sqrt(jnp.sum(r * r, axis=1, keepdims=True) + 1e-15)
    ed = jnp.exp(-d)
    mu0 = np.float32(_MU[0])
    dmu = np.float32((_MU[-1] - _MU[0]) / (N_RBF - 1))
    mu = mu0 + dmu * lax.broadcasted_iota(jnp.int32, (1, N_RBF), 1
                                          ).astype(jnp.float32)
    rbf = jnp.exp(-_BETA * (ed - mu) ** 2)
    env = 0.5 * (jnp.cos(np.float32(np.pi) / CUTOFF * d) + 1.0)
    env = env * (d < CUTOFF).astype(jnp.float32)
    out_ref[:, :N_RBF] = rbf * env
    out_ref[:, N_RBF:N_RBF + 1] = env
    out_ref[:, N_RBF + 1:] = jnp.zeros((_RB, 32 - N_RBF - 1), jnp.float32)


def _edge_basis(g):
    return pl.pallas_call(
        _edge_basis_body,
        grid=(N_EDGES // _RB,),
        in_specs=[pl.BlockSpec((2, _RB, 16), lambda j: (0, j, 0))],
        out_specs=pl.BlockSpec((_RB, 32), lambda j: (j, 0)),
        out_shape=jax.ShapeDtypeStruct((N_EDGES, 32), jnp.float32),
    )(g)


def _w_body(basis_ref, wp_ref, out_ref):
    out_ref[0, 0] = jnp.dot(basis_ref[...], wp_ref[0, 0],
                            preferred_element_type=jnp.float32)


def _w_matmul(basis, wp3):
    return pl.pallas_call(
        _w_body,
        grid=(NUM_CONV, 2, N_EDGES // _RB),
        in_specs=[
            pl.BlockSpec((_RB, 32), lambda i, c, j: (j, 0)),
            pl.BlockSpec((1, 1, 32, HALF), lambda i, c, j: (i, c, 0, 0)),
        ],
        out_specs=pl.BlockSpec((1, 1, _RB, HALF), lambda i, c, j: (i, c, j, 0)),
        out_shape=jax.ShapeDtypeStruct((NUM_CONV, 2, N_EDGES, HALF),
                                       jnp.float32),
    )(basis, wp3)


def _phi_body(s_ref, w1_ref, b1_ref, w2_ref, b2_ref, out_ref):
    h = jnp.dot(s_ref[...], w1_ref[...], preferred_element_type=jnp.float32)
    h = _silu(h + b1_ref[...])
    h = jnp.dot(h, w2_ref[...], preferred_element_type=jnp.float32) + b2_ref[...]
    out_ref[0] = h[:, :HALF]
    out_ref[1] = h[:, HALF:]


def _phi_mlp(S, p):
    return pl.pallas_call(
        _phi_body,
        grid=(N_NODES // _NB,),
        in_specs=[
            pl.BlockSpec((_NB, N_ATOM), lambda j: (j, 0)),
            pl.BlockSpec((N_ATOM, N_ATOM), lambda j: (0, 0)),
            pl.BlockSpec((1, N_ATOM), lambda j: (0, 0)),
            pl.BlockSpec((N_ATOM, N_ATOM), lambda j: (0, 0)),
            pl.BlockSpec((1, N_ATOM), lambda j: (0, 0)),
        ],
        out_specs=pl.BlockSpec((2, _NB, HALF), lambda j: (0, j, 0)),
        out_shape=jax.ShapeDtypeStruct((2, N_NODES, HALF), jnp.float32),
    )(S, p["l1"]["W"], p["l1"]["b"][None, :], p["l2"]["W"], p["l2"]["b"][None, :])


def _dense_body(s_ref, v_ref, w1_ref, b1_ref, w2_ref, b2_ref, out_ref):
    a0 = _silu(v_ref[0])
    a1 = _silu(v_ref[1])
    h = (jnp.dot(a0, w1_ref[:HALF], preferred_element_type=jnp.float32)
         + jnp.dot(a1, w1_ref[HALF:], preferred_element_type=jnp.float32)
         + b1_ref[...])
    h = _silu(h)
    h = jnp.dot(h, w2_ref[...], preferred_element_type=jnp.float32) + b2_ref[...]
    out_ref[...] = s_ref[...] + h


def _dense_update(S, v, p):
    return pl.pallas_call(
        _dense_body,
        grid=(N_NODES // _NB,),
        in_specs=[
            pl.BlockSpec((_NB, N_ATOM), lambda j: (j, 0)),
            pl.BlockSpec((2, _NB, HALF), lambda j: (0, j, 0)),
            pl.BlockSpec((N_ATOM, N_ATOM), lambda j: (0, 0)),
            pl.BlockSpec((1, N_ATOM), lambda j: (0, 0)),
            pl.BlockSpec((N_ATOM, N_ATOM), lambda j: (0, 0)),
            pl.BlockSpec((1, N_ATOM), lambda j: (0, 0)),
        ],
        out_specs=pl.BlockSpec((_NB, N_ATOM), lambda j: (j, 0)),
        out_shape=jax.ShapeDtypeStruct((N_NODES, N_ATOM), jnp.float32),
    )(S, v, p["l1"]["W"], p["l1"]["b"][None, :], p["l2"]["W"], p["l2"]["b"][None, :])


def _dense_phi_body(s_ref, v_ref, w1_ref, b1_ref, w2_ref, b2_ref,
                    m1_ref, mb1_ref, m2_ref, mb2_ref, sout_ref, phi_ref):
    a0 = _silu(v_ref[0])
    a1 = _silu(v_ref[1])
    h = (jnp.dot(a0, w1_ref[:HALF], preferred_element_type=jnp.float32)
         + jnp.dot(a1, w1_ref[HALF:], preferred_element_type=jnp.float32)
         + b1_ref[...])
    h = _silu(h)
    h = jnp.dot(h, w2_ref[...], preferred_element_type=jnp.float32) + b2_ref[...]
    s_new = s_ref[...] + h
    sout_ref[...] = s_new
    ph = jnp.dot(s_new, m1_ref[...], preferred_element_type=jnp.float32)
    ph = _silu(ph + mb1_ref[...])
    ph = jnp.dot(ph, m2_ref[...], preferred_element_type=jnp.float32) + mb2_ref[...]
    phi_ref[0] = ph[:, :HALF]
    phi_ref[1] = ph[:, HALF:]


def _dense_phi(S, v, p, mp):
    full256 = pl.BlockSpec((N_ATOM, N_ATOM), lambda j: (0, 0))
    bias = pl.BlockSpec((1, N_ATOM), lambda j: (0, 0))
    return pl.pallas_call(
        _dense_phi_body,
        grid=(N_NODES // _NB,),
        in_specs=[
            pl.BlockSpec((_NB, N_ATOM), lambda j: (j, 0)),
            pl.BlockSpec((2, _NB, HALF), lambda j: (0, j, 0)),
            full256, bias, full256, bias,
            full256, bias, full256, bias,
        ],
        out_specs=[
            pl.BlockSpec((_NB, N_ATOM), lambda j: (j, 0)),
            pl.BlockSpec((2, _NB, HALF), lambda j: (0, j, 0)),
        ],
        out_shape=[
            jax.ShapeDtypeStruct((N_NODES, N_ATOM), jnp.float32),
            jax.ShapeDtypeStruct((2, N_NODES, HALF), jnp.float32),
        ],
    )(S, v, p["l1"]["W"], p["l1"]["b"][None, :], p["l2"]["W"],
      p["l2"]["b"][None, :], mp["l1"]["W"], mp["l1"]["b"][None, :],
      mp["l2"]["W"], mp["l2"]["b"][None, :])


def _mlp2_block(x, w1, b1, w2, b2):
    h = jnp.dot(_silu(x), w1, preferred_element_type=jnp.float32) + b1
    return jnp.dot(_silu(h), w2, preferred_element_type=jnp.float32) + b2


def _heads_body(s_ref, z_ref, bdist_ref, dist_ref,
                ba1w, ba1b, ba2w, ba2b,
                bt1w, bt1b, bt2w, bt2b,
                sa1w, sa1b, sa2w, sa2b,
                t1w, t1b, t2w, t2b, t3w, t3b, t4w, t4b, t5w, t5b, t6w, t6b,
                ft1w, ft1b, ft2w, ft2b,
                bbd_ref, bba_ref, bbt_ref, scd_ref, sca_ref, sct_ref):
    S = s_ref[...]
    z = z_ref[0, 0, :]
    onehot = (z[:, None] == lax.broadcasted_iota(jnp.int32, (1, 25), 1)
              ).astype(jnp.float32)
    bbd_ref[...] = jnp.dot(onehot, bdist_ref[...],
                           preferred_element_type=jnp.float32)
    scd_ref[...] = jnp.dot(onehot, dist_ref[...],
                           preferred_element_type=jnp.float32)

    sS = _silu(S)
    bb_angle = _mlp2_block(S, ba1w[...], ba1b[...], ba2w[...], ba2b[...])
    bba_ref[...] = bb_angle
    h = (jnp.dot(sS, bt1w[:N_ATOM], preferred_element_type=jnp.float32)
         + jnp.dot(_silu(bb_angle), bt1w[N_ATOM:],
                   preferred_element_type=jnp.float32) + bt1b[...])
    bbt_ref[...] = jnp.dot(_silu(h), bt2w[...],
                           preferred_element_type=jnp.float32) + bt2b[...]
    sca_ref[...] = _mlp2_block(S, sa1w[...], sa1b[...], sa2w[...], sa2b[...])

    t = S + _mlp2_block(S, t1w[...], t1b[...], t2w[...], t2b[...])
    t = t + _mlp2_block(t, t3w[...], t3b[...], t4w[...], t4b[...])
    t = t + _mlp2_block(t, t5w[...], t5b[...], t6w[...], t6b[...])
    sct_ref[...] = _mlp2_block(t, ft1w[...], ft1b[...], ft2w[...], ft2b[...])


def _heads(S, z3, params):
    full = lambda shape: pl.BlockSpec(shape, lambda j: tuple(0 for _ in shape))
    p = params
    args = [S, z3, p["backbone_dist"], p["distance"]]
    specs = [
        pl.BlockSpec((_NB, N_ATOM), lambda j: (j, 0)),
        pl.BlockSpec((1, 1, _NB), lambda j: (j, 0, 0)),
        full((25, 3)),
        full((25, 10)),
    ]

    def add_lin(lin):
        args.append(lin["W"])
        specs.append(full(lin["W"].shape))
        args.append(lin["b"][None, :])
        specs.append(full((1, lin["b"].shape[0])))

    add_lin(p["bb_angle"]["l1"]); add_lin(p["bb_angle"]["l2"])
    add_lin(p["bb_torsion"]["l1"]); add_lin(p["bb_torsion"]["l2"])
    add_lin(p["sc_angle"]["l1"]); add_lin(p["sc_angle"]["l2"])
    for i in range(NUM_CONV):
        add_lin(p["sc_torsion"][i]["l1"]); add_lin(p["sc_torsion"][i]["l2"])
    add_lin(p["final_torsion"]["l1"]); add_lin(p["final_torsion"]["l2"])

    out_shapes = [
        jax.ShapeDtypeStruct((N_NODES, 3), jnp.float32),
        jax.ShapeDtypeStruct((N_NODES, 3), jnp.float32),
        jax.ShapeDtypeStruct((N_NODES, 3), jnp.float32),
        jax.ShapeDtypeStruct((N_NODES, 10), jnp.float32),
        jax.ShapeDtypeStruct((N_NODES, 10), jnp.float32),
        jax.ShapeDtypeStruct((N_NODES, 10), jnp.float32),
    ]
    out_specs = [pl.BlockSpec((_NB, sh.shape[1]), lambda j: (j, 0))
                 for sh in out_shapes]
    return pl.pallas_call(
        _heads_body,
        grid=(N_NODES // _NB,),
        in_specs=specs,
        out_specs=out_specs,
        out_shape=out_shapes,
    )(*args)


def kernel(cg_z, cg_xyz, CG_nbr_list, mapping, S, params):
    nbr = CG_nbr_list.astype(jnp.int32)
    src = nbr[:, 1]
    dst = nbr[:, 0]
    idx2 = jnp.stack([src, dst]).reshape(2, NS, NCHUNK, K)

    xyz16 = jnp.zeros((N_NODES, 16), jnp.float32).at[:, :3].set(cg_xyz)
    g = _geom_gather_kernel(xyz16, idx2)
    basis = _edge_basis(g)

    wp3 = jnp.stack([
        jnp.concatenate(
            [params["msg"][i]["ld"]["W"], params["msg"][i]["ld"]["b"][None, :],
             jnp.zeros((32 - N_RBF - 1, N_ATOM), jnp.float32)],
            axis=0).reshape(32, 2, HALF).transpose(1, 0, 2)
        for i in range(NUM_CONV)])
    w3 = _w_matmul(basis, wp3)

    phi = _phi_mlp(S, params["msg"][0])
    for i in range(NUM_CONV):
        v = _EDGE_CONVS[i](phi, w3, src, dst)
        if i + 1 < NUM_CONV:
            S, phi = _dense_phi(S, v, params["dense"][i], params["msg"][i + 1])
        else:
            S = _dense_update(S, v, params["dense"][i])

    z3 = cg_z.astype(jnp.int32).reshape(N_NODES // _NB, 1, _NB)
    bbd, bba, bbt, scd, sca, sct = _heads(S, z3, params)
    ic_bb = jnp.stack([bbd, bba, bbt], axis=-1)
    ic_sc = jnp.stack([scd, sca, sct], axis=-1)
    return jnp.concatenate([ic_bb, ic_sc], axis=1)


# speedup vs baseline: 1.0794x; 1.0794x over previous
import functools

import jax
import jax.numpy as jnp
import numpy as np
from jax import lax
from jax.experimental import pallas as pl
from jax.experimental.pallas import tpu as pltpu
from jax.experimental.pallas import tpu_sc as plsc

N_ATOM = 256
HALF = 128
N_RBF = 16
CUTOFF = 10.0
N_NODES = 10000
N_EDGES = 160000
NUM_CONV = 3

NC = 2
NS = 16
K = 80
EPT = N_EDGES // NS
NCHUNK = EPT // K
RPT = 624
REM_R = N_NODES - NS * RPT

_MU = np.linspace(np.exp(-CUTOFF), 1.0, N_RBF).astype(np.float32)
_BETA = np.float32((2.0 / N_RBF * (1.0 - np.exp(-CUTOFF))) ** (-2))

_sc_mesh = plsc.VectorSubcoreMesh(core_axis_name="c", subcore_axis_name="s")


@functools.partial(
    pl.kernel,
    out_type=jax.ShapeDtypeStruct((2, N_EDGES, 16), jnp.float32),
    mesh=_sc_mesh,
    compiler_params=pltpu.CompilerParams(use_tc_tiling_on_sc=False),
    scratch_types=[
        pltpu.VMEM((NCHUNK, K), jnp.int32),
        pltpu.VMEM((K, 16), jnp.float32),
        pltpu.SemaphoreType.DMA,
    ],
)
def _geom_gather_kernel(xyz_hbm, idx_hbm, out_hbm, idx_v, rows_v, sem):
    c = lax.axis_index("c")
    s = lax.axis_index("s")
    pltpu.sync_copy(idx_hbm.at[c, s], idx_v)

    def chunk(j, carry):
        pltpu.async_copy(xyz_hbm.at[idx_v.at[j]], rows_v, sem).wait()
        base = s * EPT + j * K
        pltpu.sync_copy(rows_v, out_hbm.at[c, pl.ds(base, K)])
        return carry

    lax.fori_loop(0, NCHUNK, chunk, 0)


def _make_edge_conv(ci):
  @functools.partial(
      pl.kernel,
      out_type=jax.ShapeDtypeStruct((2, N_NODES, HALF), jnp.float32),
      mesh=_sc_mesh,
      scratch_types=[
          pltpu.VMEM_SHARED((N_NODES, HALF), jnp.float32),
          [pltpu.VMEM((K,), jnp.int32)] * 2,
          [pltpu.VMEM((K,), jnp.int32)] * 2,
          [pltpu.VMEM((K, HALF), jnp.float32)] * 2,
          [pltpu.VMEM((K, HALF), jnp.float32)] * 2,
          [pltpu.SemaphoreType.DMA] * 2,
          [pltpu.SemaphoreType.DMA] * 2,
          [pltpu.SemaphoreType.DMA] * 2,
          [pltpu.SemaphoreType.DMA] * 2,
          [pltpu.SemaphoreType.DMA] * 2,
      ],
  )
  def _edge_conv_kernel(phi_hbm, w_hbm, src_hbm, dst_hbm, out_hbm,
                        acc, sidx, didx, rows, wrow,
                        gsem, wsem, scsem, sisem, disem):
    c = lax.axis_index("c")
    s = lax.axis_index("s")
    base0 = s * EPT

    zero16 = jnp.zeros((16,), jnp.float32)

    def zrow(i, carry):
        for jj in range(HALF // 16):
            rows[0][i, pl.ds(jj * 16, 16)] = zero16
        return carry

    lax.fori_loop(0, K, zrow, 0)
    base_r = s * RPT
    nfull = RPT // K
    rem = RPT - nfull * K
    for t in range(nfull):
        pltpu.sync_copy(rows[0], acc.at[pl.ds(base_r + t * K, K)])
    if rem:
        pltpu.sync_copy(rows[0].at[pl.ds(0, rem)],
                        acc.at[pl.ds(base_r + nfull * K, rem)])

    @pl.when(s == NS - 1)
    def _zero_tail():
        pltpu.sync_copy(rows[0].at[pl.ds(0, REM_R)],
                        acc.at[pl.ds(NS * RPT, REM_R)])

    plsc.subcore_barrier()

    def _issue_sidx(j, p):
        jc = jnp.minimum(j, NCHUNK - 1)
        pltpu.async_copy(src_hbm.at[pl.ds(base0 + jc * K, K)],
                         sidx[p], sisem[p])

    def _issue_didx(j, p):
        jc = jnp.minimum(j, NCHUNK - 1)
        pltpu.async_copy(dst_hbm.at[pl.ds(base0 + jc * K, K)],
                         didx[p], disem[p])

    def _issue_gather(p):
        pltpu.async_copy(phi_hbm.at[c].at[sidx[p]], rows[p], gsem[p])

    def _issue_w(j, p):
        pltpu.async_copy(w_hbm.at[c, pl.ds(base0 + j * K, K)],
                         wrow[p], wsem[p])

    def _wait(src, dst, sem):
        pltpu.make_async_copy(src, dst, sem).wait()

    def _multiply(p):
        @plsc.parallel_loop(0, K, step=1, unroll=4)
        def mul(i):
            for jj in range(HALF // 16):
                sl = pl.ds(jj * 16, 16)
                rows[p][i, sl] = rows[p][i, sl] * wrow[p][i, sl]

    def _full_iter(j, p):
        q = 1 - p
        _wait(phi_hbm.at[c].at[sidx[p]], rows[p], gsem[p])
        _wait(w_hbm.at[c, pl.ds(0, K)], wrow[p], wsem[p])

        @pl.when(j >= 1)
        def _wait_prev_scatter():
            _wait(rows[q], acc.at[didx[q]], scsem[q])

        _issue_didx(j + 1, q)
        _wait(src_hbm.at[pl.ds(0, K)], sidx[q], sisem[q])
        _issue_gather(q)
        _issue_w(j + 1, q)
        _issue_sidx(j + 2, p)
        _multiply(p)
        _wait(dst_hbm.at[pl.ds(0, K)], didx[p], disem[p])
        pltpu.async_copy(rows[p], acc.at[didx[p]], scsem[p], add=True)

    _issue_sidx(0, 0)
    _issue_didx(0, 0)
    _issue_sidx(1, 1)
    _wait(src_hbm.at[pl.ds(0, K)], sidx[0], sisem[0])
    _issue_gather(0)
    _issue_w(0, 0)

    def pair(j2, carry):
        j = j2 * 2
        _full_iter(j, 0)
        _full_iter(j + 1, 1)
        return carry

    lax.fori_loop(0, (NCHUNK - 1) // 2, pair, 0)

    _wait(phi_hbm.at[c].at[sidx[0]], rows[0], gsem[0])
    _wait(w_hbm.at[c, pl.ds(0, K)], wrow[0], wsem[0])
    _multiply(0)
    _wait(rows[1], acc.at[didx[1]], scsem[1])
    _wait(dst_hbm.at[pl.ds(0, K)], didx[0], disem[0])
    pltpu.async_copy(rows[0], acc.at[didx[0]], scsem[0], add=True)
    _wait(rows[0], acc.at[didx[0]], scsem[0])
    _wait(src_hbm.at[pl.ds(0, K)], sidx[1], sisem[1])
    plsc.subcore_barrier()
    pltpu.sync_copy(acc.at[pl.ds(base_r, RPT)],
                    out_hbm.at[c, pl.ds(base_r, RPT)])

    @pl.when(s == NS - 1)
    def _copy_tail():
        pltpu.sync_copy(acc.at[pl.ds(NS * RPT, REM_R)],
                        out_hbm.at[c, pl.ds(NS * RPT, REM_R)])

  return _edge_conv_kernel


_EDGE_CONVS = [_make_edge_conv(i) for i in range(NUM_CONV)]


_RB = 4000
_NB = 1000


def _silu(x):
    return x * jax.nn.sigmoid(x)


def _edge_basis_body(g_ref, out_ref):
    r = g_ref[0, :, :3] - g_ref[1, :, :3]
    d = jnp.sqrt(jnp.sum(r * r, axis=1, keepdims=True) + 1e-15)
    ed = jnp.exp(-d)
    mu0 = np.float32(_MU[0])
    dmu = np.float32((_MU[-1] - _MU[0]) / (N_RBF - 1))
    mu = mu0 + dmu * lax.broadcasted_iota(jnp.int32, (1, N_RBF), 1
                                          ).astype(jnp.float32)
    rbf = jnp.exp(-_BETA * (ed - mu) ** 2)
    env = 0.5 * (jnp.cos(np.float32(np.pi) / CUTOFF * d) + 1.0)
    env = env * (d < CUTOFF).astype(jnp.float32)
    out_ref[:, :N_RBF] = rbf * env
    out_ref[:, N_RBF:N_RBF + 1] = env
    out_ref[:, N_RBF + 1:] = jnp.zeros((_RB, 32 - N_RBF - 1), jnp.float32)


def _edge_basis(g):
    return pl.pallas_call(
        _edge_basis_body,
        grid=(N_EDGES // _RB,),
        in_specs=[pl.BlockSpec((2, _RB, 16), lambda j: (0, j, 0))],
        out_specs=pl.BlockSpec((_RB, 32), lambda j: (j, 0)),
        out_shape=jax.ShapeDtypeStruct((N_EDGES, 32), jnp.float32),
    )(g)


def _w_body(basis_ref, wp_ref, out_ref):
    out_ref[0] = jnp.dot(basis_ref[...], wp_ref[0],
                         preferred_element_type=jnp.float32)


def _w_matmul(basis, wp):
    return pl.pallas_call(
        _w_body,
        grid=(2, N_EDGES // _RB),
        in_specs=[
            pl.BlockSpec((_RB, 32), lambda c, j: (j, 0)),
            pl.BlockSpec((1, 32, HALF), lambda c, j: (c, 0, 0)),
        ],
        out_specs=pl.BlockSpec((1, _RB, HALF), lambda c, j: (c, j, 0)),
        out_shape=jax.ShapeDtypeStruct((2, N_EDGES, HALF), jnp.float32),
    )(basis, wp)


def _phi_body(s_ref, w1_ref, b1_ref, w2_ref, b2_ref, out_ref):
    h = jnp.dot(s_ref[...], w1_ref[...], preferred_element_type=jnp.float32)
    h = _silu(h + b1_ref[...])
    h = jnp.dot(h, w2_ref[...], preferred_element_type=jnp.float32) + b2_ref[...]
    out_ref[0] = h[:, :HALF]
    out_ref[1] = h[:, HALF:]


def _phi_mlp(S, p):
    return pl.pallas_call(
        _phi_body,
        grid=(N_NODES // _NB,),
        in_specs=[
            pl.BlockSpec((_NB, N_ATOM), lambda j: (j, 0)),
            pl.BlockSpec((N_ATOM, N_ATOM), lambda j: (0, 0)),
            pl.BlockSpec((1, N_ATOM), lambda j: (0, 0)),
            pl.BlockSpec((N_ATOM, N_ATOM), lambda j: (0, 0)),
            pl.BlockSpec((1, N_ATOM), lambda j: (0, 0)),
        ],
        out_specs=pl.BlockSpec((2, _NB, HALF), lambda j: (0, j, 0)),
        out_shape=jax.ShapeDtypeStruct((2, N_NODES, HALF), jnp.float32),
    )(S, p["l1"]["W"], p["l1"]["b"][None, :], p["l2"]["W"], p["l2"]["b"][None, :])


def _dense_body(s_ref, v_ref, w1_ref, b1_ref, w2_ref, b2_ref, out_ref):
    a0 = _silu(v_ref[0])
    a1 = _silu(v_ref[1])
    h = (jnp.dot(a0, w1_ref[:HALF], preferred_element_type=jnp.float32)
         + jnp.dot(a1, w1_ref[HALF:], preferred_element_type=jnp.float32)
         + b1_ref[...])
    h = _silu(h)
    h = jnp.dot(h, w2_ref[...], preferred_element_type=jnp.float32) + b2_ref[...]
    out_ref[...] = s_ref[...] + h


def _dense_update(S, v, p):
    return pl.pallas_call(
        _dense_body,
        grid=(N_NODES // _NB,),
        in_specs=[
            pl.BlockSpec((_NB, N_ATOM), lambda j: (j, 0)),
            pl.BlockSpec((2, _NB, HALF), lambda j: (0, j, 0)),
            pl.BlockSpec((N_ATOM, N_ATOM), lambda j: (0, 0)),
            pl.BlockSpec((1, N_ATOM), lambda j: (0, 0)),
            pl.BlockSpec((N_ATOM, N_ATOM), lambda j: (0, 0)),
            pl.BlockSpec((1, N_ATOM), lambda j: (0, 0)),
        ],
        out_specs=pl.BlockSpec((_NB, N_ATOM), lambda j: (j, 0)),
        out_shape=jax.ShapeDtypeStruct((N_NODES, N_ATOM), jnp.float32),
    )(S, v, p["l1"]["W"], p["l1"]["b"][None, :], p["l2"]["W"], p["l2"]["b"][None, :])


def _dense_phi_body(s_ref, v_ref, w1_ref, b1_ref, w2_ref, b2_ref,
                    m1_ref, mb1_ref, m2_ref, mb2_ref, sout_ref, phi_ref):
    a0 = _silu(v_ref[0])
    a1 = _silu(v_ref[1])
    h = (jnp.dot(a0, w1_ref[:HALF], preferred_element_type=jnp.float32)
         + jnp.dot(a1, w1_ref[HALF:], preferred_element_type=jnp.float32)
         + b1_ref[...])
    h = _silu(h)
    h = jnp.dot(h, w2_ref[...], preferred_element_type=jnp.float32) + b2_ref[...]
    s_new = s_ref[...] + h
    sout_ref[...] = s_new
    ph = jnp.dot(s_new, m1_ref[...], preferred_element_type=jnp.float32)
    ph = _silu(ph + mb1_ref[...])
    ph = jnp.dot(ph, m2_ref[...], preferred_element_type=jnp.float32) + mb2_ref[...]
    phi_ref[0] = ph[:, :HALF]
    phi_ref[1] = ph[:, HALF:]


def _dense_phi(S, v, p, mp):
    full256 = pl.BlockSpec((N_ATOM, N_ATOM), lambda j: (0, 0))
    bias = pl.BlockSpec((1, N_ATOM), lambda j: (0, 0))
    return pl.pallas_call(
        _dense_phi_body,
        grid=(N_NODES // _NB,),
        in_specs=[
            pl.BlockSpec((_NB, N_ATOM), lambda j: (j, 0)),
            pl.BlockSpec((2, _NB, HALF), lambda j: (0, j, 0)),
            full256, bias, full256, bias,
            full256, bias, full256, bias,
        ],
        out_specs=[
            pl.BlockSpec((_NB, N_ATOM), lambda j: (j, 0)),
            pl.BlockSpec((2, _NB, HALF), lambda j: (0, j, 0)),
        ],
        out_shape=[
            jax.ShapeDtypeStruct((N_NODES, N_ATOM), jnp.float32),
            jax.ShapeDtypeStruct((2, N_NODES, HALF), jnp.float32),
        ],
    )(S, v, p["l1"]["W"], p["l1"]["b"][None, :], p["l2"]["W"],
      p["l2"]["b"][None, :], mp["l1"]["W"], mp["l1"]["b"][None, :],
      mp["l2"]["W"], mp["l2"]["b"][None, :])


def _mlp2_block(x, w1, b1, w2, b2):
    h = jnp.dot(_silu(x), w1, preferred_element_type=jnp.float32) + b1
    return jnp.dot(_silu(h), w2, preferred_element_type=jnp.float32) + b2


def _heads_body(s_ref, z_ref, bdist_ref, dist_ref,
                ba1w, ba1b, ba2w, ba2b,
                bt1w, bt1b, bt2w, bt2b,
                sa1w, sa1b, sa2w, sa2b,
                t1w, t1b, t2w, t2b, t3w, t3b, t4w, t4b, t5w, t5b, t6w, t6b,
                ft1w, ft1b, ft2w, ft2b,
                bbd_ref, bba_ref, bbt_ref, scd_ref, sca_ref, sct_ref):
    S = s_ref[...]
    z = z_ref[0, 0, :]
    onehot = (z[:, None] == lax.broadcasted_iota(jnp.int32, (1, 25), 1)
              ).astype(jnp.float32)
    bbd_ref[...] = jnp.dot(onehot, bdist_ref[...],
                           preferred_element_type=jnp.float32)
    scd_ref[...] = jnp.dot(onehot, dist_ref[...],
                           preferred_element_type=jnp.float32)

    sS = _silu(S)
    bb_angle = _mlp2_block(S, ba1w[...], ba1b[...], ba2w[...], ba2b[...])
    bba_ref[...] = bb_angle
    h = (jnp.dot(sS, bt1w[:N_ATOM], preferred_element_type=jnp.float32)
         + jnp.dot(_silu(bb_angle), bt1w[N_ATOM:],
                   preferred_element_type=jnp.float32) + bt1b[...])
    bbt_ref[...] = jnp.dot(_silu(h), bt2w[...],
                           preferred_element_type=jnp.float32) + bt2b[...]
    sca_ref[...] = _mlp2_block(S, sa1w[...], sa1b[...], sa2w[...], sa2b[...])

    t = S + _mlp2_block(S, t1w[...], t1b[...], t2w[...], t2b[...])
    t = t + _mlp2_block(t, t3w[...], t3b[...], t4w[...], t4b[...])
    t = t + _mlp2_block(t, t5w[...], t5b[...], t6w[...], t6b[...])
    sct_ref[...] = _mlp2_block(t, ft1w[...], ft1b[...], ft2w[...], ft2b[...])


def _heads(S, z3, params):
    full = lambda shape: pl.BlockSpec(shape, lambda j: tuple(0 for _ in shape))
    p = params
    args = [S, z3, p["backbone_dist"], p["distance"]]
    specs = [
        pl.BlockSpec((_NB, N_ATOM), lambda j: (j, 0)),
        pl.BlockSpec((1, 1, _NB), lambda j: (j, 0, 0)),
        full((25, 3)),
        full((25, 10)),
    ]

    def add_lin(lin):
        args.append(lin["W"])
        specs.append(full(lin["W"].shape))
        args.append(lin["b"][None, :])
        specs.append(full((1, lin["b"].shape[0])))

    add_lin(p["bb_angle"]["l1"]); add_lin(p["bb_angle"]["l2"])
    add_lin(p["bb_torsion"]["l1"]); add_lin(p["bb_torsion"]["l2"])
    add_lin(p["sc_angle"]["l1"]); add_lin(p["sc_angle"]["l2"])
    for i in range(NUM_CONV):
        add_lin(p["sc_torsion"][i]["l1"]); add_lin(p["sc_torsion"][i]["l2"])
    add_lin(p["final_torsion"]["l1"]); add_lin(p["final_torsion"]["l2"])

    out_shapes = [
        jax.ShapeDtypeStruct((N_NODES, 3), jnp.float32),
        jax.ShapeDtypeStruct((N_NODES, 3), jnp.float32),
        jax.ShapeDtypeStruct((N_NODES, 3), jnp.float32),
        jax.ShapeDtypeStruct((N_NODES, 10), jnp.float32),
        jax.ShapeDtypeStruct((N_NODES, 10), jnp.float32),
        jax.ShapeDtypeStruct((N_NODES, 10), jnp.float32),
    ]
    out_specs = [pl.BlockSpec((_NB, sh.shape[1]), lambda j: (j, 0))
                 for sh in out_shapes]
    return pl.pallas_call(
        _heads_body,
        grid=(N_NODES // _NB,),
        in_specs=specs,
        out_specs=out_specs,
        out_shape=out_shapes,
    )(*args)


def kernel(cg_z, cg_xyz, CG_nbr_list, mapping, S, params):
    nbr = CG_nbr_list.astype(jnp.int32)
    src = nbr[:, 1]
    dst = nbr[:, 0]
    idx2 = jnp.stack([src, dst]).reshape(2, NS, NCHUNK, K)

    xyz16 = jnp.zeros((N_NODES, 16), jnp.float32).at[:, :3].set(cg_xyz)
    g = _geom_gather_kernel(xyz16, idx2)
    basis = _edge_basis(g)

    wps = [
        jnp.concatenate(
            [params["msg"][i]["ld"]["W"], params["msg"][i]["ld"]["b"][None, :],
             jnp.zeros((32 - N_RBF - 1, N_ATOM), jnp.float32)],
            axis=0).reshape(32, 2, HALF).transpose(1, 0, 2)
        for i in range(NUM_CONV)]

    phi = _phi_mlp(S, params["msg"][0])
    for i in range(NUM_CONV):
        w = _w_matmul(basis, wps[i])
        v = _EDGE_CONVS[i](phi, w, src, dst)
        if i + 1 < NUM_CONV:
            S, phi = _dense_phi(S, v, params["dense"][i], params["msg"][i + 1])
        else:
            S = _dense_update(S, v, params["dense"][i])

    z3 = cg_z.astype(jnp.int32).reshape(N_NODES // _NB, 1, _NB)
    bbd, bba, bbt, scd, sca, sct = _heads(S, z3, params)
    ic_bb = jnp.stack([bbd, bba, bbt], axis=-1)
    ic_sc = jnp.stack([scd, sca, sct], axis=-1)
    return jnp.concatenate([ic_bb, ic_sc], axis=1)
